# baseline (speedup 1.0000x reference)
import functools

import jax
import jax.numpy as jnp
from jax import lax
from jax.experimental import pallas as pl
from jax.experimental.pallas import tpu as pltpu
from jax.experimental.pallas import tpu_sc as plsc

N_FIELDS = 26
L = 20
VOCAB_P1 = 100001
VPAD = 100008
DIM = 32
B = 4096
ODIM = N_FIELDS * DIM
NLOOK = N_FIELDS * L
PAD = 528
GCHUNK = 104
NGCHUNK = NLOOK // GCHUNK
NC = 2
NS = 16
NW = NC * NS
ROWS_PER_W = B // NW
INV_L = 1.0 / L


def _emb_body(x_hbm, off_hbm, tbl_hbm, out_hbm,
              offbuf, idxbuf, idxa, idxb, gbufa, gbufb, obuf, gsem):
    wid = lax.axis_index("s") * NC + lax.axis_index("c")
    row0 = wid * ROWS_PER_W

    pltpu.sync_copy(off_hbm, offbuf)

    def row_body(i, _):
        row = row0 + i
        return 0
        pltpu.sync_copy(
            x_hbm.at[pl.ds(pl.multiple_of(row * NLOOK, 8), NLOOK)],
            idxbuf.at[pl.ds(0, NLOOK)])
        def add_body(j, _):
            s = pl.ds(pl.multiple_of(j * 16, 16), 16)
            a = (idxbuf[s] + offbuf[s]) * 8
            idxa[s] = a
            idxb[s] = a + 1
            return 0
        lax.fori_loop(0, PAD // 16, add_body, 0)
        handles = []
        for j in range(NGCHUNK):
            sl = pl.ds(j * GCHUNK, GCHUNK)
            handles.append(
                pltpu.async_copy(tbl_hbm.at[idxa.at[sl]], gbufa.at[sl], gsem))
            handles.append(
                pltpu.async_copy(tbl_hbm.at[idxb.at[sl]], gbufb.at[sl], gsem))
        for h in handles:
            h.wait()
        def field_body(f, _):
            base = pl.multiple_of(f * L, L)
            a0 = gbufa[base, :]
            a1 = gbufb[base, :]
            for l in range(1, L):
                a0 = a0 + gbufa[base + l, :]
                a1 = a1 + gbufb[base + l, :]
            o = pl.multiple_of(f * DIM, DIM)
            obuf[pl.ds(o, 16)] = a0 * INV_L
            obuf[pl.ds(o + 16, 16)] = a1 * INV_L
            return 0
        lax.fori_loop(0, N_FIELDS, field_body, 0)
        pltpu.sync_copy(
            obuf,
            out_hbm.at[pl.ds(pl.multiple_of(row * ODIM, 8), ODIM)])
        return 0

    lax.fori_loop(0, ROWS_PER_W, row_body, 0)


@jax.jit
def _emb(x, off, tbl):
    mesh = plsc.VectorSubcoreMesh(core_axis_name="c", subcore_axis_name="s")
    f = pl.kernel(
        _emb_body,
        mesh=mesh,
        out_type=jax.ShapeDtypeStruct((B * ODIM,), jnp.float32),
        scratch_types=[
            pltpu.VMEM((PAD,), jnp.int32),
            pltpu.VMEM((PAD,), jnp.int32),
            pltpu.VMEM((PAD,), jnp.int32),
            pltpu.VMEM((PAD,), jnp.int32),
            pltpu.VMEM((NLOOK, 16), jnp.float32),
            pltpu.VMEM((NLOOK, 16), jnp.float32),
            pltpu.VMEM((ODIM,), jnp.float32),
            pltpu.SemaphoreType.DMA,
        ],
        compiler_params=pltpu.CompilerParams(use_tc_tiling_on_sc=False),
    )
    return f(x, off, tbl)


def kernel(x, tables):
    tbl = jnp.pad(tables, ((0, 0), (0, VPAD - VOCAB_P1), (0, 0)))
    tbl = tbl.reshape(N_FIELDS * VPAD * 2, 16)
    off = jnp.repeat(
        jnp.arange(N_FIELDS, dtype=jnp.int32) * jnp.int32(VPAD), L)
    off = jnp.concatenate([off, jnp.zeros((PAD - NLOOK,), jnp.int32)])
    out = _emb(x.reshape(-1), off, tbl)
    return out.reshape(B, ODIM)


# speedup vs baseline: 1.5056x; 1.5056x over previous
import functools

import jax
import jax.numpy as jnp
from jax import lax
from jax.experimental import pallas as pl
from jax.experimental.pallas import tpu as pltpu
from jax.experimental.pallas import tpu_sc as plsc

N_FIELDS = 26
L = 20
VOCAB_P1 = 100001
VPAD = 100008
DIM = 32
B = 4096
ODIM = N_FIELDS * DIM
NLOOK = N_FIELDS * L
PAD = 528
GCHUNK = 104
NGCHUNK = NLOOK // GCHUNK
NC = 2
NS = 16
NW = NC * NS
ROWS_PER_W = B // NW
INV_L = 1.0 / L


def _emb_body(x_hbm, off_hbm, tbl_hbm, out_hbm,
              offbuf, idxbuf, idxa, idxb, gbufa, gbufb, obuf,
              isem0, isem1, gsem0, gsem1, osem0, osem1):
    wid = lax.axis_index("s") * NC + lax.axis_index("c")
    row0 = wid * ROWS_PER_W
    isem = (isem0, isem1)
    gsem = (gsem0, gsem1)
    osem = (osem0, osem1)

    pltpu.sync_copy(off_hbm, offbuf)

    def fire_idx(row, s):
        return pltpu.async_copy(
            x_hbm.at[pl.ds(pl.multiple_of(row * NLOOK, 8), NLOOK)],
            idxbuf.at[s].at[pl.ds(0, NLOOK)], isem[s])

    def wait_idx(s):
        pltpu.make_async_copy(
            x_hbm.at[pl.ds(0, NLOOK)],
            idxbuf.at[s].at[pl.ds(0, NLOOK)], isem[s]).wait()

    def build_idx(s):
        def add_body(j, _):
            sl = pl.ds(pl.multiple_of(j * 16, 16), 16)
            a = (idxbuf.at[s][sl] + offbuf[sl]) * 8
            idxa.at[s][sl] = a
            idxb.at[s][sl] = a + 1
            return 0
        lax.fori_loop(0, PAD // 16, add_body, 0)

    def fire_gathers(s):
        for j in range(NGCHUNK):
            sl = pl.ds(j * GCHUNK, GCHUNK)
            pltpu.async_copy(
                tbl_hbm.at[idxa.at[s].at[sl]], gbufa.at[s].at[sl], gsem[s])
            pltpu.async_copy(
                tbl_hbm.at[idxb.at[s].at[sl]], gbufb.at[s].at[sl], gsem[s])

    def wait_gathers(s):
        for j in range(NGCHUNK):
            sl = pl.ds(j * GCHUNK, GCHUNK)
            pltpu.make_async_copy(
                tbl_hbm.at[pl.ds(0, GCHUNK)], gbufa.at[s].at[sl],
                gsem[s]).wait()
            pltpu.make_async_copy(
                tbl_hbm.at[pl.ds(0, GCHUNK)], gbufb.at[s].at[sl],
                gsem[s]).wait()

    def wait_out(s):
        pltpu.make_async_copy(
            obuf.at[s], out_hbm.at[pl.ds(0, ODIM)], osem[s]).wait()

    def reduce_and_out(row, s, first):
        @pl.when(jnp.logical_not(first))
        def _():
            wait_out(s)

        def field_body(f, _):
            base = pl.multiple_of(f * L, L)
            a0 = gbufa.at[s][base, :]
            a1 = gbufb.at[s][base, :]
            for l in range(1, L):
                a0 = a0 + gbufa.at[s][base + l, :]
                a1 = a1 + gbufb.at[s][base + l, :]
            o = pl.multiple_of(f * DIM, DIM)
            obuf.at[s][pl.ds(o, 16)] = a0 * INV_L
            obuf.at[s][pl.ds(o + 16, 16)] = a1 * INV_L
            return 0
        lax.fori_loop(0, N_FIELDS, field_body, 0)
        pltpu.async_copy(
            obuf.at[s],
            out_hbm.at[pl.ds(pl.multiple_of(row * ODIM, 8), ODIM)], osem[s])

    fire_idx(row0, 0)

    def body(k, _):
        r0 = row0 + 2 * k
        wait_idx(0)
        build_idx(0)
        fire_gathers(0)
        fire_idx(r0 + 1, 1)
        @pl.when(k > 0)
        def _():
            wait_gathers(1)
            reduce_and_out(r0 - 1, 1, k == 1)
        wait_idx(1)
        build_idx(1)
        fire_gathers(1)
        @pl.when(k < ROWS_PER_W // 2 - 1)
        def _():
            fire_idx(r0 + 2, 0)
        wait_gathers(0)
        reduce_and_out(r0, 0, k == 0)
        return 0

    lax.fori_loop(0, ROWS_PER_W // 2, body, 0)

    wait_gathers(1)
    reduce_and_out(row0 + ROWS_PER_W - 1, 1, jnp.bool_(False))
    wait_out(0)
    wait_out(1)


@jax.jit
def _emb(x, off, tbl):
    mesh = plsc.VectorSubcoreMesh(core_axis_name="c", subcore_axis_name="s")
    f = pl.kernel(
        _emb_body,
        mesh=mesh,
        out_type=jax.ShapeDtypeStruct((B * ODIM,), jnp.float32),
        scratch_types=[
            pltpu.VMEM((PAD,), jnp.int32),
            pltpu.VMEM((2, PAD), jnp.int32),
            pltpu.VMEM((2, PAD), jnp.int32),
            pltpu.VMEM((2, PAD), jnp.int32),
            pltpu.VMEM((2, NLOOK, 16), jnp.float32),
            pltpu.VMEM((2, NLOOK, 16), jnp.float32),
            pltpu.VMEM((2, ODIM), jnp.float32),
            pltpu.SemaphoreType.DMA,
            pltpu.SemaphoreType.DMA,
            pltpu.SemaphoreType.DMA,
            pltpu.SemaphoreType.DMA,
            pltpu.SemaphoreType.DMA,
            pltpu.SemaphoreType.DMA,
        ],
        compiler_params=pltpu.CompilerParams(use_tc_tiling_on_sc=False),
    )
    return f(x, off, tbl)


def kernel(x, tables):
    tbl = jnp.pad(tables, ((0, 0), (0, VPAD - VOCAB_P1), (0, 128 - DIM)))
    tbl = tbl.reshape(N_FIELDS * VPAD * 8, 16)
    off = jnp.repeat(
        jnp.arange(N_FIELDS, dtype=jnp.int32) * jnp.int32(VPAD), L)
    off = jnp.concatenate([off, jnp.zeros((PAD - NLOOK,), jnp.int32)])
    out = _emb(x.reshape(-1), off, tbl)
    return out.reshape(B, ODIM)
